# baseline (speedup 1.0000x reference)
import jax
import jax.numpy as jnp
from jax import lax
from jax.experimental import pallas as pl
from jax.experimental.pallas import tpu as pltpu, tpu_sc as plsc

N = 50000
E = 800000
G = 512
EPS = 1e-5

NC = 2
NS = 16
ER = E // 128
EPR = 6400
NB = 4
NB64 = 5
NPS = N // NS
BR = 2000
GRID = N // BR

_mesh = plsc.VectorSubcoreMesh(core_axis_name="c", subcore_axis_name="s")
_sc_params = pltpu.CompilerParams(use_tc_tiling_on_sc=False)


def _f32(*shape):
    return jax.ShapeDtypeStruct(shape, jnp.float32)


def _deg_body(dst_hbm, zeros_hbm, ones_hbm, out_hbm, dacc, idx_v, ones_v, sem):
    c = lax.axis_index("c")
    s = lax.axis_index("s")
    rbase = s * NPS
    pltpu.sync_copy(zeros_hbm.at[pl.ds(rbase, NPS)], dacc.at[pl.ds(rbase, NPS)])
    pltpu.sync_copy(ones_hbm, ones_v)
    plsc.subcore_barrier()
    e0 = c * (EPR // 2) + s * (EPR // 2 // NS)

    def step(g, carry):
        base = e0 + g * NB
        pltpu.sync_copy(dst_hbm.at[pl.ds(base, NB)], idx_v)
        descs = [pltpu.async_copy(ones_v, dacc.at[idx_v.at[b]], sem, add=True)
                 for b in range(NB)]
        for d in descs:
            d.wait()
        return carry

    lax.fori_loop(0, EPR // 2 // NS // NB, step, 0)
    plsc.subcore_barrier()
    pltpu.sync_copy(dacc.at[pl.ds(rbase, NPS)], out_hbm.at[c, pl.ds(rbase, NPS)])


_deg_kernel = pl.kernel(
    _deg_body,
    out_type=_f32(2, N, 16),
    mesh=_mesh,
    scratch_types=[
        pltpu.VMEM_SHARED((N + 8, 16), jnp.float32),
        pltpu.VMEM((NB, 128), jnp.int32),
        pltpu.VMEM((128, 16), jnp.float32),
        pltpu.SemaphoreType.DMA,
    ],
    compiler_params=_sc_params,
)


def _agg16_body(xs_hbm, zeros_hbm, src_hbm, dst_hbm, out_hbm,
                acc, src_v, dst_v, rows_v, sem, sem2):
    c = lax.axis_index("c")
    s = lax.axis_index("s")
    rbase = s * NPS

    @pl.when(c == 0)
    def _():
        pltpu.sync_copy(xs_hbm.at[pl.ds(rbase, NPS)], acc.at[pl.ds(rbase, NPS)])

    @pl.when(c == 1)
    def _():
        pltpu.sync_copy(zeros_hbm.at[pl.ds(rbase, NPS)], acc.at[pl.ds(rbase, NPS)])

    plsc.subcore_barrier()
    e0 = c * (EPR // 2) + s * (EPR // 2 // NS)

    def step(g, carry):
        base = e0 + g * NB
        pltpu.sync_copy(src_hbm.at[pl.ds(base, NB)], src_v)
        pltpu.sync_copy(dst_hbm.at[pl.ds(base, NB)], dst_v)
        gds = [pltpu.async_copy(xs_hbm.at[src_v.at[b]], rows_v.at[b], sem)
               for b in range(NB)]
        for d in gds:
            d.wait()
        sds = [pltpu.async_copy(rows_v.at[b], acc.at[dst_v.at[b]],
                                sem2, add=True) for b in range(NB)]
        for d in sds:
            d.wait()
        return carry

    lax.fori_loop(0, EPR // 2 // NS // NB, step, 0)
    plsc.subcore_barrier()
    pltpu.sync_copy(acc.at[pl.ds(rbase, NPS)], out_hbm.at[c, pl.ds(rbase, NPS)])


_agg16_kernel = pl.kernel(
    _agg16_body,
    out_type=_f32(2, N, 16),
    mesh=_mesh,
    scratch_types=[
        pltpu.VMEM_SHARED((N + 8, 16), jnp.float32),
        pltpu.VMEM((NB, 128), jnp.int32),
        pltpu.VMEM((NB, 128), jnp.int32),
        pltpu.VMEM((NB, 128, 16), jnp.float32),
        pltpu.SemaphoreType.DMA,
        pltpu.SemaphoreType.DMA,
    ],
    compiler_params=_sc_params,
)


def _agg64_body(lo_hbm, hi_hbm, src_hbm, dst_hbm, out_lo, out_hi,
                acc, src_v, dst_v, rows_v, sem, sem2):
    c = lax.axis_index("c")
    s = lax.axis_index("s")
    rbase = s * NPS

    @pl.when(c == 0)
    def _():
        pltpu.sync_copy(lo_hbm.at[pl.ds(rbase, NPS)], acc.at[pl.ds(rbase, NPS)])

    @pl.when(c == 1)
    def _():
        pltpu.sync_copy(hi_hbm.at[pl.ds(rbase, NPS)], acc.at[pl.ds(rbase, NPS)])

    plsc.subcore_barrier()
    e0 = s * (EPR // NS)

    def edge_loop(table):
        def step(g, carry):
            base = e0 + g * NB64
            pltpu.sync_copy(src_hbm.at[pl.ds(base, NB64)], src_v)
            pltpu.sync_copy(dst_hbm.at[pl.ds(base, NB64)], dst_v)
            gds = [pltpu.async_copy(table.at[src_v.at[b]], rows_v.at[b], sem)
                   for b in range(NB64)]
            for d in gds:
                d.wait()
            sds = [pltpu.async_copy(rows_v.at[b], acc.at[dst_v.at[b]],
                                    sem2, add=True) for b in range(NB64)]
            for d in sds:
                d.wait()
            return carry

        lax.fori_loop(0, EPR // NS // NB64, step, 0)

    @pl.when(c == 0)
    def _():
        edge_loop(lo_hbm)

    @pl.when(c == 1)
    def _():
        edge_loop(hi_hbm)
    plsc.subcore_barrier()

    @pl.when(c == 0)
    def _():
        pltpu.sync_copy(acc.at[pl.ds(rbase, NPS)], out_lo.at[pl.ds(rbase, NPS)])

    @pl.when(c == 1)
    def _():
        pltpu.sync_copy(acc.at[pl.ds(rbase, NPS)], out_hi.at[pl.ds(rbase, NPS)])


_agg64_kernel = pl.kernel(
    _agg64_body,
    out_type=(_f32(N, 32), _f32(N, 32)),
    mesh=_mesh,
    scratch_types=[
        pltpu.VMEM_SHARED((N + 8, 32), jnp.float32),
        pltpu.VMEM((NB64, 128), jnp.int32),
        pltpu.VMEM((NB64, 128), jnp.int32),
        pltpu.VMEM((NB64, 128, 32), jnp.float32),
        pltpu.SemaphoreType.DMA,
        pltpu.SemaphoreType.DMA,
    ],
    compiler_params=_sc_params,
)


def _pool_body(h_hbm, b2d_hbm, zg_hbm, out_hbm, pacc, idx_v, rows_v):
    c = lax.axis_index("c")
    s = lax.axis_index("s")
    gbase = s * (G // NS)
    pltpu.sync_copy(zg_hbm.at[pl.ds(gbase, G // NS)],
                    pacc.at[pl.ds(gbase, G // NS)])
    plsc.subcore_barrier()
    w = c * NS + s
    r0 = w * 97 + jnp.minimum(w, 21)
    cnt = 97 + (w < 21).astype(jnp.int32)

    def step(j, carry):
        pltpu.sync_copy(b2d_hbm.at[j], idx_v)
        pltpu.sync_copy(h_hbm.at[pl.ds(j * 16, 16)], rows_v)
        pltpu.sync_copy(rows_v, pacc.at[idx_v], add=True)
        return carry

    lax.fori_loop(r0, r0 + cnt, step, 0)
    plsc.subcore_barrier()
    pltpu.sync_copy(pacc.at[pl.ds(gbase, G // NS)],
                    out_hbm.at[c, pl.ds(gbase, G // NS)])


_pool_kernel = pl.kernel(
    _pool_body,
    out_type=_f32(2, G, 64),
    mesh=_mesh,
    scratch_types=[
        pltpu.VMEM_SHARED((G, 64), jnp.float32),
        pltpu.VMEM((16,), jnp.int32),
        pltpu.VMEM((16, 64), jnp.float32),
    ],
    compiler_params=_sc_params,
)


def _dinv_from(dp):
    deg = dp[0] + dp[1] + 1.0
    return lax.rsqrt(deg)


def _t1a_body(x_ref, dp_ref, xs_ref):
    dinv = _dinv_from(dp_ref[...])
    xs_ref[...] = x_ref[...] * dinv


def _t2_body(p_ref, dp_ref, w1_ref, b1_ref, g1_ref, be1_ref, w2_ref,
             lo_ref, hi_ref):
    dinv = _dinv_from(dp_ref[...])
    dinv_col = dinv[:, 0:1]
    p = p_ref[...]
    y = (p[0] + p[1]) * dinv
    h1 = jnp.maximum(
        jnp.dot(y, w1_ref[...], preferred_element_type=jnp.float32,
                precision=lax.Precision.HIGHEST) + b1_ref[...], 0.0)
    h1 = h1 * g1_ref[...] + be1_ref[...]
    hs = jnp.dot(h1, w2_ref[...], preferred_element_type=jnp.float32,
                 precision=lax.Precision.HIGHEST) * dinv_col
    lo_ref[...] = hs[:, :32]
    hi_ref[...] = hs[:, 32:]


def _tmid_body(lo_in, hi_in, dp_ref, b_ref, w_ref, lo_ref, hi_ref):
    dinv = _dinv_from(dp_ref[...])
    dinv_col = dinv[:, 0:1]
    agg = jnp.concatenate([lo_in[...], hi_in[...]], axis=1)
    h = jnp.maximum(agg * dinv_col + b_ref[...], 0.0)
    hs = jnp.dot(h, w_ref[...], preferred_element_type=jnp.float32,
                 precision=lax.Precision.HIGHEST) * dinv_col
    lo_ref[...] = hs[:, :32]
    hi_ref[...] = hs[:, 32:]


def _t5_body(lo_in, hi_in, dp_ref, b_ref, out_ref):
    dinv = _dinv_from(dp_ref[...])
    dinv_col = dinv[:, 0:1]
    agg = jnp.concatenate([lo_in[...], hi_in[...]], axis=1)
    h = jnp.maximum(agg * dinv_col + b_ref[...], 0.0)
    lane = lax.broadcasted_iota(jnp.int32, h.shape, 1)
    out_ref[...] = jnp.where(lane == 63, 1.0, h)


def _t6_body(p_ref, wl_ref, bl_ref, g3_ref, be3_ref, wl2_ref, bl2_ref, out_ref):
    p = p_ref[...]
    ps = p[0] + p[1]
    cnt = jnp.maximum(ps[:, 63:64], 1.0)
    mean = ps / cnt
    z = jnp.maximum(
        jnp.dot(mean, wl_ref[...], preferred_element_type=jnp.float32,
                precision=lax.Precision.HIGHEST) + bl_ref[...], 0.0)
    z = z * g3_ref[...] + be3_ref[...]
    logits = jnp.dot(z, wl2_ref[...], preferred_element_type=jnp.float32,
                     precision=lax.Precision.HIGHEST) + bl2_ref[...]
    lane = lax.broadcasted_iota(jnp.int32, logits.shape, 1)
    m = jnp.where(lane < 4, logits, -1e30)
    mx = jnp.max(m, axis=1, keepdims=True)
    lse = jnp.log(jnp.sum(jnp.exp(m - mx), axis=1, keepdims=True))
    out_ref[...] = m - mx - lse


def _row_spec(w):
    return pl.BlockSpec((BR, w), lambda i: (i, 0))


def _full_spec(*shape):
    n = len(shape)
    return pl.BlockSpec(shape, lambda i: (0,) * n)


_dp_spec = pl.BlockSpec((2, BR, 16), lambda i: (0, i, 0))

_t1a = pl.pallas_call(
    _t1a_body, grid=(GRID,),
    in_specs=[_row_spec(16), _dp_spec],
    out_specs=_row_spec(16),
    out_shape=_f32(N, 16),
)

_t2 = pl.pallas_call(
    _t2_body, grid=(GRID,),
    in_specs=[_dp_spec, _dp_spec, _full_spec(16, 64), _full_spec(1, 64),
              _full_spec(1, 64), _full_spec(1, 64), _full_spec(64, 64)],
    out_specs=(_row_spec(32), _row_spec(32)),
    out_shape=(_f32(N, 32), _f32(N, 32)),
)

_tmid = pl.pallas_call(
    _tmid_body, grid=(GRID,),
    in_specs=[_row_spec(32), _row_spec(32), _dp_spec, _full_spec(1, 64),
              _full_spec(64, 64)],
    out_specs=(_row_spec(32), _row_spec(32)),
    out_shape=(_f32(N, 32), _f32(N, 32)),
)

_t5 = pl.pallas_call(
    _t5_body, grid=(GRID,),
    in_specs=[_row_spec(32), _row_spec(32), _dp_spec, _full_spec(1, 64)],
    out_specs=_row_spec(64),
    out_shape=_f32(N, 64),
)

_t6 = pl.pallas_call(
    _t6_body, grid=(1,),
    in_specs=[_full_spec(2, G, 64), _full_spec(64, 64), _full_spec(1, 64),
              _full_spec(1, 64), _full_spec(1, 64), _full_spec(64, 8),
              _full_spec(1, 8)],
    out_specs=_full_spec(G, 8),
    out_shape=_f32(G, 8),
)


def _pad2(a, r, c):
    return jnp.pad(a, ((0, r - a.shape[0]), (0, c - a.shape[1])))


def _padrow(v, c):
    return jnp.pad(v, (0, c - v.shape[0])).reshape(1, c)


def kernel(x, edge_index, batch, W1, b1, W2, b2, W3, b3, W4, b4,
           g1, be1, g3, be3, Wl, bl, Wl2, bl2):
    pad = EPR * 128 - E
    src2d = jnp.concatenate(
        [edge_index[0], jnp.zeros((pad,), jnp.int32)]).reshape(EPR, 128)
    dst2d = jnp.concatenate(
        [edge_index[1], jnp.full((pad,), N, jnp.int32)]).reshape(EPR, 128)
    batch2d = batch.reshape(3125, 16)

    inv_bn = 1.0 / jnp.sqrt(1.0 + EPS)
    W1p = _pad2(W1, 16, 64)
    W2p, W3p, W4p = (_pad2(w, 64, 64) for w in (W2, W3, W4))
    Wlp = _pad2(Wl, 64, 64)
    Wl2p = _pad2(Wl2, 64, 8)
    b1p, b2p, b3p, b4p = (_padrow(b, 64) for b in (b1, b2, b3, b4))
    blp = _padrow(bl, 64)
    bl2p = _padrow(bl2, 8)
    g1e = _padrow(g1 * inv_bn, 64)
    be1p = _padrow(be1, 64)
    g3e = _padrow(g3 * inv_bn, 64)
    be3p = _padrow(be3, 64)

    zerosN16 = jnp.zeros((N, 16), jnp.float32)
    ones128 = jnp.ones((128, 16), jnp.float32)
    zerosG = jnp.zeros((G, 64), jnp.float32)

    degp = _deg_kernel(dst2d, zerosN16, ones128)
    xs = _t1a(x, degp)
    p1 = _agg16_kernel(xs, zerosN16, src2d, dst2d)
    lo2, hi2 = _t2(p1, degp, W1p, b1p, g1e, be1p, W2p)
    alo2, ahi2 = _agg64_kernel(lo2, hi2, src2d, dst2d)
    lo3, hi3 = _tmid(alo2, ahi2, degp, b2p, W3p)
    alo3, ahi3 = _agg64_kernel(lo3, hi3, src2d, dst2d)
    lo4, hi4 = _tmid(alo3, ahi3, degp, b3p, W4p)
    alo4, ahi4 = _agg64_kernel(lo4, hi4, src2d, dst2d)
    hpool = _t5(alo4, ahi4, degp, b4p)
    pooled = _pool_kernel(hpool, batch2d, zerosG)
    out = _t6(pooled, Wlp, blp, g3e, be3p, Wl2p, bl2p)
    return out[:, :4]


# speedup vs baseline: 1.2636x; 1.2636x over previous
import jax
import jax.numpy as jnp
from jax import lax
from jax.experimental import pallas as pl
from jax.experimental.pallas import tpu as pltpu, tpu_sc as plsc

N = 50000
E = 800000
G = 512
EPS = 1e-5

NC = 2
NS = 16
ER = E // 128
EPR = 6272
NB = 4
NB64 = 4
NPS = N // NS
BR = 2000
GRID = N // BR

_mesh = plsc.VectorSubcoreMesh(core_axis_name="c", subcore_axis_name="s")
_sc_params = pltpu.CompilerParams(use_tc_tiling_on_sc=False)


def _f32(*shape):
    return jax.ShapeDtypeStruct(shape, jnp.float32)


def _deg_body(dst_hbm, zeros_hbm, ones_hbm, out_hbm, dacc, idx_v, ones_v, sem):
    c = lax.axis_index("c")
    s = lax.axis_index("s")
    rbase = s * NPS
    pltpu.sync_copy(zeros_hbm.at[pl.ds(rbase, NPS)], dacc.at[pl.ds(rbase, NPS)])
    pltpu.sync_copy(ones_hbm, ones_v)
    plsc.subcore_barrier()
    e0 = c * (EPR // 2) + s * (EPR // 2 // NS)

    def step(g, carry):
        base = e0 + g * NB
        pltpu.sync_copy(dst_hbm.at[pl.ds(base, NB)], idx_v)
        descs = [pltpu.async_copy(ones_v, dacc.at[idx_v.at[b]], sem, add=True)
                 for b in range(NB)]
        for d in descs:
            d.wait()
        return carry

    lax.fori_loop(0, EPR // 2 // NS // NB, step, 0)
    plsc.subcore_barrier()
    pltpu.sync_copy(dacc.at[pl.ds(rbase, NPS)], out_hbm.at[c, pl.ds(rbase, NPS)])


_deg_kernel = pl.kernel(
    _deg_body,
    out_type=_f32(2, N, 16),
    mesh=_mesh,
    scratch_types=[
        pltpu.VMEM_SHARED((N + 8, 16), jnp.float32),
        pltpu.VMEM((NB, 128), jnp.int32),
        pltpu.VMEM((128, 16), jnp.float32),
        pltpu.SemaphoreType.DMA,
    ],
    compiler_params=_sc_params,
)


def _agg16_body(xs_hbm, zeros_hbm, src_hbm, dst_hbm, out_hbm,
                acc, src_v, dst_v, rows_v, sem, sem2):
    c = lax.axis_index("c")
    s = lax.axis_index("s")
    rbase = s * NPS

    @pl.when(c == 0)
    def _():
        pltpu.sync_copy(xs_hbm.at[pl.ds(rbase, NPS)], acc.at[pl.ds(rbase, NPS)])

    @pl.when(c == 1)
    def _():
        pltpu.sync_copy(zeros_hbm.at[pl.ds(rbase, NPS)], acc.at[pl.ds(rbase, NPS)])

    plsc.subcore_barrier()
    e0 = c * (EPR // 2) + s * (EPR // 2 // NS)

    def step(g, carry):
        base = e0 + g * NB
        pltpu.sync_copy(src_hbm.at[pl.ds(base, NB)], src_v)
        pltpu.sync_copy(dst_hbm.at[pl.ds(base, NB)], dst_v)
        gds = [pltpu.async_copy(xs_hbm.at[src_v.at[b]], rows_v.at[b], sem)
               for b in range(NB)]
        for d in gds:
            d.wait()
        sds = [pltpu.async_copy(rows_v.at[b], acc.at[dst_v.at[b]],
                                sem2, add=True) for b in range(NB)]
        for d in sds:
            d.wait()
        return carry

    lax.fori_loop(0, EPR // 2 // NS // NB, step, 0)
    plsc.subcore_barrier()
    pltpu.sync_copy(acc.at[pl.ds(rbase, NPS)], out_hbm.at[c, pl.ds(rbase, NPS)])


_agg16_kernel = pl.kernel(
    _agg16_body,
    out_type=_f32(2, N, 16),
    mesh=_mesh,
    scratch_types=[
        pltpu.VMEM_SHARED((N + 8, 16), jnp.float32),
        pltpu.VMEM((NB, 128), jnp.int32),
        pltpu.VMEM((NB, 128), jnp.int32),
        pltpu.VMEM((NB, 128, 16), jnp.float32),
        pltpu.SemaphoreType.DMA,
        pltpu.SemaphoreType.DMA,
    ],
    compiler_params=_sc_params,
)


def _agg64_body(lo_hbm, hi_hbm, src_hbm, dst_hbm, out_lo, out_hi,
                acc, src_v, dst_v, rows_v, sem, sem2):
    c = lax.axis_index("c")
    s = lax.axis_index("s")
    rbase = s * NPS

    @pl.when(c == 0)
    def _():
        pltpu.sync_copy(lo_hbm.at[pl.ds(rbase, NPS)], acc.at[pl.ds(rbase, NPS)])

    @pl.when(c == 1)
    def _():
        pltpu.sync_copy(hi_hbm.at[pl.ds(rbase, NPS)], acc.at[pl.ds(rbase, NPS)])

    plsc.subcore_barrier()
    e0 = s * (EPR // NS)

    def edge_loop(table):
        def step(g, carry):
            base = e0 + g * NB64
            pltpu.sync_copy(src_hbm.at[pl.ds(base, NB64)], src_v)
            pltpu.sync_copy(dst_hbm.at[pl.ds(base, NB64)], dst_v)
            gds = [pltpu.async_copy(table.at[src_v.at[b]], rows_v.at[b], sem)
                   for b in range(NB64)]
            for d in gds:
                d.wait()
            sds = [pltpu.async_copy(rows_v.at[b], acc.at[dst_v.at[b]],
                                    sem2, add=True) for b in range(NB64)]
            for d in sds:
                d.wait()
            return carry

        lax.fori_loop(0, EPR // NS // NB64, step, 0)

    @pl.when(c == 0)
    def _():
        edge_loop(lo_hbm)

    @pl.when(c == 1)
    def _():
        edge_loop(hi_hbm)
    plsc.subcore_barrier()

    @pl.when(c == 0)
    def _():
        pltpu.sync_copy(acc.at[pl.ds(rbase, NPS)], out_lo.at[pl.ds(rbase, NPS)])

    @pl.when(c == 1)
    def _():
        pltpu.sync_copy(acc.at[pl.ds(rbase, NPS)], out_hi.at[pl.ds(rbase, NPS)])


_agg64_kernel = pl.kernel(
    _agg64_body,
    out_type=(_f32(N, 32), _f32(N, 32)),
    mesh=_mesh,
    scratch_types=[
        pltpu.VMEM_SHARED((N + 8, 32), jnp.float32),
        pltpu.VMEM((NB64, 128), jnp.int32),
        pltpu.VMEM((NB64, 128), jnp.int32),
        pltpu.VMEM((NB64, 128, 32), jnp.float32),
        pltpu.SemaphoreType.DMA,
        pltpu.SemaphoreType.DMA,
    ],
    compiler_params=_sc_params,
)


def _pool_body(h_hbm, b2d_hbm, zg_hbm, out_hbm, pacc, idx_v, rows_v):
    c = lax.axis_index("c")
    s = lax.axis_index("s")
    gbase = s * (G // NS)
    pltpu.sync_copy(zg_hbm.at[pl.ds(gbase, G // NS)],
                    pacc.at[pl.ds(gbase, G // NS)])
    plsc.subcore_barrier()
    w = c * NS + s
    r0 = w * 97 + jnp.minimum(w, 21)
    cnt = 97 + (w < 21).astype(jnp.int32)

    def step(j, carry):
        pltpu.sync_copy(b2d_hbm.at[j], idx_v)
        pltpu.sync_copy(h_hbm.at[pl.ds(j * 16, 16)], rows_v)
        pltpu.sync_copy(rows_v, pacc.at[idx_v], add=True)
        return carry

    lax.fori_loop(r0, r0 + cnt, step, 0)
    plsc.subcore_barrier()
    pltpu.sync_copy(pacc.at[pl.ds(gbase, G // NS)],
                    out_hbm.at[c, pl.ds(gbase, G // NS)])


_pool_kernel = pl.kernel(
    _pool_body,
    out_type=_f32(2, G, 64),
    mesh=_mesh,
    scratch_types=[
        pltpu.VMEM_SHARED((G, 64), jnp.float32),
        pltpu.VMEM((16,), jnp.int32),
        pltpu.VMEM((16, 64), jnp.float32),
    ],
    compiler_params=_sc_params,
)


def _dinv_from(dp):
    deg = dp[0] + dp[1] + 1.0
    return lax.rsqrt(deg)


def _t1a_body(x_ref, dp_ref, xs_ref):
    dinv = _dinv_from(dp_ref[...])
    xs_ref[...] = x_ref[...] * dinv


def _t2_body(p_ref, dp_ref, w1_ref, b1_ref, g1_ref, be1_ref, w2_ref,
             lo_ref, hi_ref):
    dinv = _dinv_from(dp_ref[...])
    dinv_col = dinv[:, 0:1]
    p = p_ref[...]
    y = (p[0] + p[1]) * dinv
    h1 = jnp.maximum(
        jnp.dot(y, w1_ref[...], preferred_element_type=jnp.float32,
                precision=lax.Precision.HIGHEST) + b1_ref[...], 0.0)
    h1 = h1 * g1_ref[...] + be1_ref[...]
    hs = jnp.dot(h1, w2_ref[...], preferred_element_type=jnp.float32,
                 precision=lax.Precision.HIGHEST) * dinv_col
    lo_ref[...] = hs[:, :32]
    hi_ref[...] = hs[:, 32:]


def _tmid_body(lo_in, hi_in, dp_ref, b_ref, w_ref, lo_ref, hi_ref):
    dinv = _dinv_from(dp_ref[...])
    dinv_col = dinv[:, 0:1]
    agg = jnp.concatenate([lo_in[...], hi_in[...]], axis=1)
    h = jnp.maximum(agg * dinv_col + b_ref[...], 0.0)
    hs = jnp.dot(h, w_ref[...], preferred_element_type=jnp.float32,
                 precision=lax.Precision.HIGHEST) * dinv_col
    lo_ref[...] = hs[:, :32]
    hi_ref[...] = hs[:, 32:]


def _t5_body(lo_in, hi_in, dp_ref, b_ref, out_ref):
    dinv = _dinv_from(dp_ref[...])
    dinv_col = dinv[:, 0:1]
    agg = jnp.concatenate([lo_in[...], hi_in[...]], axis=1)
    h = jnp.maximum(agg * dinv_col + b_ref[...], 0.0)
    lane = lax.broadcasted_iota(jnp.int32, h.shape, 1)
    out_ref[...] = jnp.where(lane == 63, 1.0, h)


def _t6_body(p_ref, wl_ref, bl_ref, g3_ref, be3_ref, wl2_ref, bl2_ref, out_ref):
    p = p_ref[...]
    ps = p[0] + p[1]
    cnt = jnp.maximum(ps[:, 63:64], 1.0)
    mean = ps / cnt
    z = jnp.maximum(
        jnp.dot(mean, wl_ref[...], preferred_element_type=jnp.float32,
                precision=lax.Precision.HIGHEST) + bl_ref[...], 0.0)
    z = z * g3_ref[...] + be3_ref[...]
    logits = jnp.dot(z, wl2_ref[...], preferred_element_type=jnp.float32,
                     precision=lax.Precision.HIGHEST) + bl2_ref[...]
    lane = lax.broadcasted_iota(jnp.int32, logits.shape, 1)
    m = jnp.where(lane < 4, logits, -1e30)
    mx = jnp.max(m, axis=1, keepdims=True)
    lse = jnp.log(jnp.sum(jnp.exp(m - mx), axis=1, keepdims=True))
    out_ref[...] = m - mx - lse


def _row_spec(w):
    return pl.BlockSpec((BR, w), lambda i: (i, 0))


def _full_spec(*shape):
    n = len(shape)
    return pl.BlockSpec(shape, lambda i: (0,) * n)


_dp_spec = pl.BlockSpec((2, BR, 16), lambda i: (0, i, 0))

_t1a = pl.pallas_call(
    _t1a_body, grid=(GRID,),
    in_specs=[_row_spec(16), _dp_spec],
    out_specs=_row_spec(16),
    out_shape=_f32(N, 16),
)

_t2 = pl.pallas_call(
    _t2_body, grid=(GRID,),
    in_specs=[_dp_spec, _dp_spec, _full_spec(16, 64), _full_spec(1, 64),
              _full_spec(1, 64), _full_spec(1, 64), _full_spec(64, 64)],
    out_specs=(_row_spec(32), _row_spec(32)),
    out_shape=(_f32(N, 32), _f32(N, 32)),
)

_tmid = pl.pallas_call(
    _tmid_body, grid=(GRID,),
    in_specs=[_row_spec(32), _row_spec(32), _dp_spec, _full_spec(1, 64),
              _full_spec(64, 64)],
    out_specs=(_row_spec(32), _row_spec(32)),
    out_shape=(_f32(N, 32), _f32(N, 32)),
)

_t5 = pl.pallas_call(
    _t5_body, grid=(GRID,),
    in_specs=[_row_spec(32), _row_spec(32), _dp_spec, _full_spec(1, 64)],
    out_specs=_row_spec(64),
    out_shape=_f32(N, 64),
)

_t6 = pl.pallas_call(
    _t6_body, grid=(1,),
    in_specs=[_full_spec(2, G, 64), _full_spec(64, 64), _full_spec(1, 64),
              _full_spec(1, 64), _full_spec(1, 64), _full_spec(64, 8),
              _full_spec(1, 8)],
    out_specs=_full_spec(G, 8),
    out_shape=_f32(G, 8),
)


def _pad2(a, r, c):
    return jnp.pad(a, ((0, r - a.shape[0]), (0, c - a.shape[1])))


def _padrow(v, c):
    return jnp.pad(v, (0, c - v.shape[0])).reshape(1, c)


def kernel(x, edge_index, batch, W1, b1, W2, b2, W3, b3, W4, b4,
           g1, be1, g3, be3, Wl, bl, Wl2, bl2):
    pad = EPR * 128 - E
    src2d = jnp.concatenate(
        [edge_index[0], jnp.zeros((pad,), jnp.int32)]).reshape(EPR, 128)
    dst2d = jnp.concatenate(
        [edge_index[1], jnp.full((pad,), N, jnp.int32)]).reshape(EPR, 128)
    batch2d = batch.reshape(3125, 16)

    inv_bn = 1.0 / jnp.sqrt(1.0 + EPS)
    W1p = _pad2(W1, 16, 64)
    W2p, W3p, W4p = (_pad2(w, 64, 64) for w in (W2, W3, W4))
    Wlp = _pad2(Wl, 64, 64)
    Wl2p = _pad2(Wl2, 64, 8)
    b1p, b2p, b3p, b4p = (_padrow(b, 64) for b in (b1, b2, b3, b4))
    blp = _padrow(bl, 64)
    bl2p = _padrow(bl2, 8)
    g1e = _padrow(g1 * inv_bn, 64)
    be1p = _padrow(be1, 64)
    g3e = _padrow(g3 * inv_bn, 64)
    be3p = _padrow(be3, 64)

    zerosN16 = jnp.zeros((N, 16), jnp.float32)
    ones128 = jnp.ones((128, 16), jnp.float32)
    zerosG = jnp.zeros((G, 64), jnp.float32)

    degp = _deg_kernel(dst2d, zerosN16, ones128)
    xs = _t1a(x, degp)
    p1 = _agg16_kernel(xs, zerosN16, src2d, dst2d)
    lo2, hi2 = _t2(p1, degp, W1p, b1p, g1e, be1p, W2p)
    alo2, ahi2 = _agg64_kernel(lo2, hi2, src2d, dst2d)
    lo3, hi3 = _tmid(alo2, ahi2, degp, b2p, W3p)
    alo3, ahi3 = _agg64_kernel(lo3, hi3, src2d, dst2d)
    lo4, hi4 = _tmid(alo3, ahi3, degp, b3p, W4p)
    alo4, ahi4 = _agg64_kernel(lo4, hi4, src2d, dst2d)
    hpool = _t5(alo4, ahi4, degp, b4p)
    pooled = _pool_kernel(hpool, batch2d, zerosG)
    out = _t6(pooled, Wlp, blp, g3e, be3p, Wl2p, bl2p)
    return out[:, :4]
